# Initial kernel scaffold; baseline (speedup 1.0000x reference)
import jax
import jax.numpy as jnp
from jax.experimental import pallas as pl


def kernel(items, table):
    raise NotImplementedError("write your pallas kernel here")


# speedup vs baseline: 5.0717x; 5.0717x over previous
import functools

import jax
import jax.numpy as jnp
from jax import lax
from jax.experimental import pallas as pl
from jax.experimental.pallas import tpu as pltpu
from jax.experimental.pallas import tpu_sc as plsc

NUM_WORKERS = 32
CHUNK = 1600


def _make_gather(B, D, b_per_w, n_chunks):
    mesh = plsc.VectorSubcoreMesh(core_axis_name="c", subcore_axis_name="s")

    @functools.partial(
        pl.kernel,
        out_type=jax.ShapeDtypeStruct((B, D), jnp.float32),
        mesh=mesh,
        scratch_types=[
            pltpu.VMEM((CHUNK,), jnp.int32),
            pltpu.VMEM((CHUNK, D), jnp.float32),
            pltpu.SemaphoreType.DMA,
        ],
        compiler_params=pltpu.CompilerParams(use_tc_tiling_on_sc=False),
    )
    def gather_kernel(idx_hbm, table_hbm, out_hbm, idx_v, rows_v, sem):
        wid = lax.axis_index("s") * 2 + lax.axis_index("c")
        base = wid * b_per_w

        def body(i, carry):
            off = base + i * CHUNK
            pltpu.sync_copy(idx_hbm.at[pl.ds(off, CHUNK)], idx_v)
            pltpu.async_copy(table_hbm.at[idx_v], rows_v, sem).wait()
            pltpu.sync_copy(rows_v, out_hbm.at[pl.ds(off, CHUNK)])
            return carry

        lax.fori_loop(0, n_chunks, body, 0)

    return gather_kernel


def kernel(items, table):
    B0, H = items.shape
    D = table.shape[1]
    idx = items.reshape(-1).astype(jnp.int32)
    B = B0 * H
    b_per_w = B // NUM_WORKERS
    n_chunks = b_per_w // CHUNK
    out = _make_gather(B, D, b_per_w, n_chunks)(idx, table.astype(jnp.float32))
    return out.reshape(B0, H, D)


# speedup vs baseline: 5.0907x; 1.0037x over previous
import functools

import jax
import jax.numpy as jnp
from jax import lax
from jax.experimental import pallas as pl
from jax.experimental.pallas import tpu as pltpu
from jax.experimental.pallas import tpu_sc as plsc

NUM_WORKERS = 32
CHUNK = 1600


def _make_gather(B, D, b_per_w, n_chunks):
    mesh = plsc.VectorSubcoreMesh(core_axis_name="c", subcore_axis_name="s")

    @functools.partial(
        pl.kernel,
        out_type=jax.ShapeDtypeStruct((B, D), jnp.float32),
        mesh=mesh,
        scratch_types=[
            pltpu.VMEM((CHUNK,), jnp.int32),
            pltpu.VMEM((CHUNK,), jnp.int32),
            pltpu.VMEM((CHUNK, D), jnp.float32),
            pltpu.VMEM((CHUNK, D), jnp.float32),
            pltpu.SemaphoreType.DMA,
            pltpu.SemaphoreType.DMA,
            pltpu.SemaphoreType.DMA,
            pltpu.SemaphoreType.DMA,
            pltpu.SemaphoreType.DMA,
        ],
        compiler_params=pltpu.CompilerParams(use_tc_tiling_on_sc=False),
    )
    def gather_kernel(idx_hbm, table_hbm, out_hbm, idx_v0, idx_v1, rows_v0,
                      rows_v1, sem_i0, sem_i1, sem_g, sem_o0, sem_o1):
        wid = lax.axis_index("s") * 2 + lax.axis_index("c")
        base = wid * b_per_w
        idx_v = (idx_v0, idx_v1)
        rows_v = (rows_v0, rows_v1)
        sem_i = (sem_i0, sem_i1)
        sem_o = (sem_o0, sem_o1)

        def idx_copy(i, p):
            return pltpu.make_async_copy(
                idx_hbm.at[pl.ds(base + i * CHUNK, CHUNK)], idx_v[p], sem_i[p])

        def gather_copy(p):
            return pltpu.make_async_copy(
                table_hbm.at[idx_v[p]], rows_v[p], sem_g)

        def out_copy(i, p):
            return pltpu.make_async_copy(
                rows_v[p], out_hbm.at[pl.ds(base + i * CHUNK, CHUNK)],
                sem_o[p])

        idx_copy(0, 0).start()
        idx_copy(1, 1).start()
        idx_copy(0, 0).wait()
        gather_copy(0).start()
        gather_copy(0).wait()
        out_copy(0, 0).start()
        idx_copy(2, 0).start()
        idx_copy(1, 1).wait()
        gather_copy(1).start()
        gather_copy(1).wait()
        out_copy(1, 1).start()
        idx_copy(3, 1).start()
        out_copy(0, 0).wait()
        idx_copy(2, 0).wait()
        gather_copy(0).start()

        def body(g, carry):
            i = 2 * g + 2
            gather_copy(0).wait()
            out_copy(i, 0).start()
            idx_copy(i + 2, 0).start()
            out_copy(i - 1, 1).wait()
            idx_copy(i + 1, 1).wait()
            gather_copy(1).start()
            gather_copy(1).wait()
            out_copy(i + 1, 1).start()
            idx_copy(i + 3, 1).start()
            out_copy(i, 0).wait()
            idx_copy(i + 2, 0).wait()
            gather_copy(0).start()
            return carry

        lax.fori_loop(0, (n_chunks - 4) // 2, body, 0)

        i = n_chunks - 2
        gather_copy(0).wait()
        out_copy(i, 0).start()
        out_copy(i - 1, 1).wait()
        idx_copy(i + 1, 1).wait()
        gather_copy(1).start()
        gather_copy(1).wait()
        out_copy(i + 1, 1).start()
        out_copy(i, 0).wait()
        out_copy(i + 1, 1).wait()

    return gather_kernel


def kernel(items, table):
    B0, H = items.shape
    D = table.shape[1]
    idx = items.reshape(-1).astype(jnp.int32)
    B = B0 * H
    b_per_w = B // NUM_WORKERS
    n_chunks = b_per_w // CHUNK
    out = _make_gather(B, D, b_per_w, n_chunks)(idx, table.astype(jnp.float32))
    return out.reshape(B0, H, D)
